# baseline (speedup 1.0000x reference)
import functools

import jax
import jax.numpy as jnp
from jax import lax
from jax.experimental import pallas as pl
from jax.experimental.pallas import tpu as pltpu
from jax.experimental.pallas import tpu_sc as plsc

N = 10000
E = 320000
D_IN = 128
H = 256
B = 128
ADME = 30
NPAD = 10240
HALF = 128
ROWB = 1000
NEG_INF = float("-inf")

EPS_AGG = E // 16
AGG_CHUNKS = EPS_AGG // 128
AGG_REM = EPS_AGG - AGG_CHUNKS * 128
EPS_HIST = E // 32
HIST_CHUNKS = EPS_HIST // 128
HIST_REM = EPS_HIST - HIST_CHUNKS * 128


@functools.lru_cache(maxsize=None)
def _sc_hist_kernel():
    mesh = plsc.VectorSubcoreMesh(core_axis_name="c", subcore_axis_name="s")
    return functools.partial(
        pl.kernel, mesh=mesh,
        out_type=jax.ShapeDtypeStruct((2 * NPAD,), jnp.float32),
        scratch_types=[
            pltpu.VMEM((640,), jnp.float32),
            pltpu.VMEM((128,), jnp.float32),
            pltpu.VMEM((16,), jnp.float32),
            pltpu.VMEM((128,), jnp.int32),
            pltpu.VMEM((16,), jnp.int32),
            pltpu.VMEM_SHARED((NPAD,), jnp.float32),
        ],
    )(_sc_hist_body)


def _sc_hist(dst):
    return _sc_hist_kernel()(dst)


def _sc_hist_body(dst_hbm, out_hbm, zbuf, ones_v, ones16_v, idx_v, idx16_v, acc):
    c = lax.axis_index("c")
    s = lax.axis_index("s")
    wid = s * 2 + c

    def _fill(i, _):
        zbuf[pl.ds(i * 16, 16)] = jnp.zeros((16,), jnp.float32)
        return 0
    lax.fori_loop(0, 40, _fill, 0)
    for k in range(8):
        ones_v[pl.ds(k * 16, 16)] = jnp.ones((16,), jnp.float32)
    ones16_v[...] = jnp.ones((16,), jnp.float32)

    pltpu.sync_copy(zbuf, acc.at[pl.ds(s * 640, 640)])
    plsc.subcore_barrier()

    base = wid * EPS_HIST
    def _chunk(j, _):
        pltpu.sync_copy(dst_hbm.at[pl.ds(base + j * 128, 128)], idx_v)
        pltpu.sync_copy(ones_v, acc.at[idx_v], add=True)
        return 0
    lax.fori_loop(0, HIST_CHUNKS, _chunk, 0)
    pltpu.sync_copy(dst_hbm.at[pl.ds(base + HIST_CHUNKS * 128, 16)], idx16_v)
    pltpu.sync_copy(ones16_v, acc.at[idx16_v], add=True)
    plsc.subcore_barrier()

    pltpu.sync_copy(acc.at[pl.ds(s * 640, 640)], zbuf)
    pltpu.sync_copy(zbuf, out_hbm.at[pl.ds(c * NPAD + s * 640, 640)])


GEDGES = 256
NGROUPS = E // GEDGES
GPT = 78
PR = 5120


@functools.lru_cache(maxsize=None)
def _sc_agg_kernel():
    mesh = plsc.VectorSubcoreMesh(core_axis_name="c", subcore_axis_name="s")
    return functools.partial(
        pl.kernel, mesh=mesh,
        out_type=jax.ShapeDtypeStruct((2 * N, HALF), jnp.float32),
        scratch_types=[
            pltpu.VMEM((128, HALF), jnp.float32),
            pltpu.VMEM((128, HALF), jnp.float32),
            pltpu.VMEM((128, HALF), jnp.float32),
            pltpu.VMEM((128, HALF), jnp.float32),
            pltpu.VMEM((128, HALF), jnp.float32),
            pltpu.VMEM((GEDGES,), jnp.int32),
            pltpu.VMEM((GEDGES,), jnp.int32),
            pltpu.VMEM((128,), jnp.int32),
            pltpu.VMEM((128,), jnp.int32),
            pltpu.VMEM((128,), jnp.int32),
            pltpu.VMEM((128,), jnp.int32),
            pltpu.VMEM_SHARED((PR + 8, HALF), jnp.float32),
            pltpu.SemaphoreType.DMA,
            pltpu.SemaphoreType.DMA,
            pltpu.SemaphoreType.DMA,
            pltpu.SemaphoreType.DMA,
        ],
    )(_sc_agg_body)


def _sc_agg(hcat, src, dst):
    return _sc_agg_kernel()(hcat, src, dst)


def _sc_agg_body(hcat_hbm, src_hbm, dst_hbm, out_hbm,
                 stage, ma0, ma1, mb0, mb1, sba, sbb,
                 da0, da1, db0, db1, acc,
                 sem_a0, sem_a1, sem_b0, sem_b1):
    c = lax.axis_index("c")
    s = lax.axis_index("s")
    rbase = s * 320
    gbase = s * GPT
    cn = c * N

    def _mv(lo, off, nrows, into_acc):
        if into_acc:
            pltpu.sync_copy(hcat_hbm.at[pl.ds(cn + lo + off, nrows)],
                            stage.at[pl.ds(0, nrows)])
            pltpu.sync_copy(stage.at[pl.ds(0, nrows)],
                            acc.at[pl.ds(off, nrows)])
        else:
            pltpu.sync_copy(acc.at[pl.ds(off, nrows)],
                            stage.at[pl.ds(0, nrows)])
            pltpu.sync_copy(stage.at[pl.ds(0, nrows)],
                            out_hbm.at[pl.ds(cn + lo + off, nrows)])

    def _copy_stripe(p, into_acc):
        lo = p * PR
        if p == 0:
            _mv(lo, rbase, 128, into_acc)
            _mv(lo, rbase + 128, 128, into_acc)
            _mv(lo, rbase + 256, 64, into_acc)
        else:
            @pl.when(s < 15)
            def _():
                _mv(lo, rbase, 128, into_acc)
                _mv(lo, rbase + 128, 128, into_acc)
                _mv(lo, rbase + 256, 64, into_acc)

            @pl.when(s == 15)
            def _():
                _mv(lo, rbase, 80, into_acc)

    def _remap(p, dstb):
        for k in range(8):
            v = dstb[pl.ds(k * 16, 16)]
            if p == 0:
                dstb[pl.ds(k * 16, 16)] = jnp.where(v < PR, v, PR)
            else:
                dstb[pl.ds(k * 16, 16)] = jnp.where(v >= PR, v - PR, PR)

    def _run_group(p, gi, srcb, d0, d1, m0, m1, sem0, sem1):
        eo = gi * GEDGES
        pltpu.sync_copy(src_hbm.at[pl.ds(eo, GEDGES)], srcb)
        pltpu.sync_copy(dst_hbm.at[pl.ds(eo, 128)], d0)
        pltpu.sync_copy(dst_hbm.at[pl.ds(eo + 128, 128)], d1)
        for k in range(GEDGES // 16):
            srcb[pl.ds(k * 16, 16)] = srcb[pl.ds(k * 16, 16)] + cn
        _remap(p, d0)
        _remap(p, d1)
        h0 = pltpu.async_copy(hcat_hbm.at[srcb.at[pl.ds(0, 128)]], m0, sem0)
        h1 = pltpu.async_copy(hcat_hbm.at[srcb.at[pl.ds(128, 128)]], m1, sem1)
        h0.wait()
        sc0 = pltpu.async_copy(m0, acc.at[d0], sem0, add=True)
        h1.wait()
        sc1 = pltpu.async_copy(m1, acc.at[d1], sem1, add=True)
        return sc0, sc1

    for p in (0, 1):
        _copy_stripe(p, True)
        plsc.subcore_barrier()

        def _pair(i, _):
            ha = _run_group(p, gbase + 2 * i, sba, da0, da1, ma0, ma1,
                            sem_a0, sem_a1)
            hb = _run_group(p, gbase + 2 * i + 1, sbb, db0, db1, mb0, mb1,
                            sem_b0, sem_b1)
            ha[0].wait()
            ha[1].wait()
            hb[0].wait()
            hb[1].wait()
            return 0
        lax.fori_loop(0, GPT // 2, _pair, 0)

        @pl.when(s < 2)
        def _():
            hx = _run_group(p, NGROUPS - 2 + s, sba, da0, da1, ma0, ma1,
                            sem_a0, sem_a1)
            hx[0].wait()
            hx[1].wait()

        plsc.subcore_barrier()
        _copy_stripe(p, False)


def _mm_body(x_ref, w_ref, deg_ref, out_ref):
    dinv = lax.rsqrt(deg_ref[...])
    out_ref[...] = jnp.dot(x_ref[...], w_ref[...],
                           preferred_element_type=jnp.float32) * dinv


def _mm(xin, w, degcol):
    k = xin.shape[1]
    return pl.pallas_call(
        _mm_body,
        grid=(20,),
        in_specs=[
            pl.BlockSpec((ROWB, k), lambda i: (i % 10, 0)),
            pl.BlockSpec((k, HALF), lambda i: (0, i // 10)),
            pl.BlockSpec((ROWB, 1), lambda i: (i % 10, 0)),
        ],
        out_specs=pl.BlockSpec((ROWB, HALF), lambda i: (i, 0)),
        out_shape=jax.ShapeDtypeStruct((2 * N, HALF), jnp.float32),
    )(xin, w, degcol)


def _stats_body(a0_ref, a1_ref, deg_ref, b_ref, gpre_ref, s1_ref, s2_ref):
    dinv = lax.rsqrt(deg_ref[...])
    g = jnp.concatenate([a0_ref[...], a1_ref[...]], axis=1) * dinv + b_ref[...]
    gpre_ref[...] = g

    @pl.when(pl.program_id(0) == 0)
    def _():
        s1_ref[...] = jnp.zeros_like(s1_ref)
        s2_ref[...] = jnp.zeros_like(s2_ref)

    s1_ref[...] += jnp.sum(g, axis=0, keepdims=True)
    s2_ref[...] += jnp.sum(g * g, axis=0, keepdims=True)


def _stats(accf, degcol, brow):
    return pl.pallas_call(
        _stats_body,
        grid=(10,),
        in_specs=[
            pl.BlockSpec((ROWB, HALF), lambda i: (i, 0)),
            pl.BlockSpec((ROWB, HALF), lambda i: (i + 10, 0)),
            pl.BlockSpec((ROWB, 1), lambda i: (i, 0)),
            pl.BlockSpec((1, H), lambda i: (0, 0)),
        ],
        out_specs=[
            pl.BlockSpec((ROWB, H), lambda i: (i, 0)),
            pl.BlockSpec((1, H), lambda i: (0, 0)),
            pl.BlockSpec((1, H), lambda i: (0, 0)),
        ],
        out_shape=[
            jax.ShapeDtypeStruct((N, H), jnp.float32),
            jax.ShapeDtypeStruct((1, H), jnp.float32),
            jax.ShapeDtypeStruct((1, H), jnp.float32),
        ],
    )(accf, accf, degcol, brow)


def _norm_body(g_ref, s1_ref, s2_ref, ga_ref, be_ref, prev_ref, rs_ref,
               out_ref):
    m = s1_ref[...] * (1.0 / N)
    v = s2_ref[...] * (1.0 / N) - m * m
    rstd = lax.rsqrt(v + 1e-5)
    y = (g_ref[...] - m) * rstd * ga_ref[...] + be_ref[...]
    y = jnp.maximum(y, 0.0)
    out_ref[...] = y + prev_ref[...] * rs_ref[...]


def _norm(gpre, s1, s2, garow, berow, xprev, rscale):
    return pl.pallas_call(
        _norm_body,
        grid=(10,),
        in_specs=[
            pl.BlockSpec((ROWB, H), lambda i: (i, 0)),
            pl.BlockSpec((1, H), lambda i: (0, 0)),
            pl.BlockSpec((1, H), lambda i: (0, 0)),
            pl.BlockSpec((1, H), lambda i: (0, 0)),
            pl.BlockSpec((1, H), lambda i: (0, 0)),
            pl.BlockSpec((ROWB, H), lambda i: (i, 0)),
            pl.BlockSpec((1, 1), lambda i: (0, 0)),
        ],
        out_specs=pl.BlockSpec((ROWB, H), lambda i: (i, 0)),
        out_shape=jax.ShapeDtypeStruct((N, H), jnp.float32),
    )(gpre, s1, s2, garow, berow, xprev, rscale)


def _gelu(x):
    return 0.5 * x * (1.0 + lax.erf(x * 0.7071067811865476))


def _pool1_body(x3_ref, b_ref, gw1_ref, gb1_ref, gw2_ref, gb2_ref,
                lw_ref, lb_ref,
                gate_ref, cnt_ref, s1_ref, sl_ref, gm_ref, m_ref):
    x3 = x3_ref[...]
    t = _gelu(jnp.dot(x3, gw1_ref[...],
                      preferred_element_type=jnp.float32) + gb1_ref[...])
    gate = jnp.dot(t, gw2_ref[...],
                   preferred_element_type=jnp.float32) + gb2_ref[...]
    gate_ref[...] = gate
    loc = _gelu(jnp.dot(x3, lw_ref[...],
                        preferred_element_type=jnp.float32) + lb_ref[...])

    bcol = b_ref[...]
    io = lax.broadcasted_iota(jnp.int32, (ROWB, B), 1)
    ob = bcol == io
    ohf = ob.astype(jnp.float32)
    ones_col = jnp.ones((ROWB, 1), jnp.float32)
    dn = (((0,), (0,)), ((), ()))
    cntc = lax.dot_general(ohf, ones_col, dn,
                           preferred_element_type=jnp.float32)
    s1c = lax.dot_general(ohf, x3, dn,
                          preferred_element_type=jnp.float32)
    slc = lax.dot_general(ohf, loc, dn,
                          preferred_element_type=jnp.float32)
    gmc = jnp.max(jnp.where(ob, gate, NEG_INF), axis=0, keepdims=True)

    @pl.when(pl.program_id(0) == 0)
    def _():
        cnt_ref[...] = jnp.zeros_like(cnt_ref)
        s1_ref[...] = jnp.zeros_like(s1_ref)
        sl_ref[...] = jnp.zeros_like(sl_ref)
        gm_ref[...] = jnp.full_like(gm_ref, NEG_INF)
        m_ref[...] = jnp.full_like(m_ref, NEG_INF)

    cnt_ref[...] += cntc
    s1_ref[...] += s1c
    sl_ref[...] += slc
    gm_ref[...] = jnp.maximum(gm_ref[...], gmc)

    bmin = jnp.min(bcol)
    bmax = jnp.max(bcol)
    rio = lax.broadcasted_iota(jnp.int32, (B, 1), 0)

    def _mb(bi, _):
        mask = bcol == bi
        mrow = jnp.max(jnp.where(mask, x3, NEG_INF), axis=0, keepdims=True)
        cur = m_ref[...]
        m_ref[...] = jnp.where(rio == bi, jnp.maximum(cur, mrow), cur)
        return 0

    lax.fori_loop(bmin, bmax + 1, _mb, 0)


def _pool1(x3, batchcol, gw1, gb1, gw2, gb2, lw, lb):
    return pl.pallas_call(
        _pool1_body,
        grid=(10,),
        in_specs=[
            pl.BlockSpec((ROWB, H), lambda i: (i, 0)),
            pl.BlockSpec((ROWB, 1), lambda i: (i, 0)),
            pl.BlockSpec((H, 128), lambda i: (0, 0)),
            pl.BlockSpec((1, 128), lambda i: (0, 0)),
            pl.BlockSpec((128, 1), lambda i: (0, 0)),
            pl.BlockSpec((1, 1), lambda i: (0, 0)),
            pl.BlockSpec((H, 128), lambda i: (0, 0)),
            pl.BlockSpec((1, 128), lambda i: (0, 0)),
        ],
        out_specs=[
            pl.BlockSpec((ROWB, 1), lambda i: (i, 0)),
            pl.BlockSpec((B, 1), lambda i: (0, 0)),
            pl.BlockSpec((B, H), lambda i: (0, 0)),
            pl.BlockSpec((B, 128), lambda i: (0, 0)),
            pl.BlockSpec((1, B), lambda i: (0, 0)),
            pl.BlockSpec((B, H), lambda i: (0, 0)),
        ],
        out_shape=[
            jax.ShapeDtypeStruct((N, 1), jnp.float32),
            jax.ShapeDtypeStruct((B, 1), jnp.float32),
            jax.ShapeDtypeStruct((B, H), jnp.float32),
            jax.ShapeDtypeStruct((B, 128), jnp.float32),
            jax.ShapeDtypeStruct((1, B), jnp.float32),
            jax.ShapeDtypeStruct((B, H), jnp.float32),
        ],
    )(x3, batchcol, gw1, gb1, gw2, gb2, lw, lb)


def _pool2_body(x3_ref, gate_ref, b_ref, gm_ref, den_ref, z_ref):
    x3 = x3_ref[...]
    gate = gate_ref[...]
    bcol = b_ref[...]
    io = lax.broadcasted_iota(jnp.int32, (ROWB, B), 1)
    ob = bcol == io
    ohf = ob.astype(jnp.float32)
    gmb = jnp.sum(jnp.where(ob, gm_ref[...], 0.0), axis=1, keepdims=True)
    e = jnp.exp(gate - gmb)
    dn = (((0,), (0,)), ((), ()))
    denc = lax.dot_general(ohf, e, dn,
                           preferred_element_type=jnp.float32)
    zc = lax.dot_general(ohf * e, x3, dn,
                         preferred_element_type=jnp.float32)

    @pl.when(pl.program_id(0) == 0)
    def _():
        den_ref[...] = jnp.zeros_like(den_ref)
        z_ref[...] = jnp.zeros_like(z_ref)

    den_ref[...] += denc
    z_ref[...] += zc


def _pool2(x3, gate, batchcol, gm):
    return pl.pallas_call(
        _pool2_body,
        grid=(10,),
        in_specs=[
            pl.BlockSpec((ROWB, H), lambda i: (i, 0)),
            pl.BlockSpec((ROWB, 1), lambda i: (i, 0)),
            pl.BlockSpec((ROWB, 1), lambda i: (i, 0)),
            pl.BlockSpec((1, B), lambda i: (0, 0)),
        ],
        out_specs=[
            pl.BlockSpec((B, 1), lambda i: (0, 0)),
            pl.BlockSpec((B, H), lambda i: (0, 0)),
        ],
        out_shape=[
            jax.ShapeDtypeStruct((B, 1), jnp.float32),
            jax.ShapeDtypeStruct((B, H), jnp.float32),
        ],
    )(x3, gate, batchcol, gm)


def _head_body(cnt_ref, s1_ref, m_ref, z_ref, den_ref, sl_ref, adme_ref,
               w1_ref, b1_ref, w2_ref, b2_ref, w3_ref, b3_ref,
               w4_ref, b4_ref, w5_ref, b5_ref, out_ref, comb):
    c = jnp.maximum(cnt_ref[...], 1.0)
    comb[:, 0:256] = s1_ref[...] / c
    comb[:, 256:512] = m_ref[...]
    comb[:, 512:768] = z_ref[...] / den_ref[...]
    comb[:, 768:896] = sl_ref[...] / c
    comb[:, 896:1024] = jnp.concatenate(
        [adme_ref[...], jnp.zeros((B, 98), jnp.float32)], axis=1)
    h = comb[...]
    h = jnp.maximum(jnp.dot(h, w1_ref[...],
                            preferred_element_type=jnp.float32)
                    + b1_ref[...], 0.0)
    h = jnp.maximum(jnp.dot(h, w2_ref[...],
                            preferred_element_type=jnp.float32)
                    + b2_ref[...], 0.0)
    h = jnp.maximum(jnp.dot(h, w3_ref[...],
                            preferred_element_type=jnp.float32)
                    + b3_ref[...], 0.0)
    h = jnp.maximum(jnp.dot(h, w4_ref[...],
                            preferred_element_type=jnp.float32)
                    + b4_ref[...], 0.0)
    out_ref[...] = jnp.dot(h, w5_ref[...],
                           preferred_element_type=jnp.float32) + b5_ref[...]


def _head(cnt, s1, m, z, den, sl, adme, w1p, b1, w2, b2, w3, b3, w4, b4,
          w5, b5):
    return pl.pallas_call(
        _head_body,
        out_shape=jax.ShapeDtypeStruct((B, 1), jnp.float32),
        scratch_shapes=[pltpu.VMEM((B, 1024), jnp.float32)],
    )(cnt, s1, m, z, den, sl, adme, w1p, b1, w2, b2, w3, b3, w4, b4, w5, b5)


def kernel(x, edge_index, batch, adme_features, W1, b1, W2, b2, W3, b3,
           g1, be1, g2, be2, g3, be3, gW1, gb1, gW2, gb2, lW, lb,
           hW1, hb1, hW2, hb2, hW3, hb3, hW4, hb4, hW5, hb5):
    src = edge_index[0]
    dst = edge_index[1]

    hist = _sc_hist(dst)
    degcol = (hist[:NPAD][:N] + hist[NPAD:][:N] + 1.0).reshape(N, 1)

    batchcol = batch.reshape(N, 1)
    row = lambda v: v.reshape(1, -1)

    wst = jnp.stack([jnp.pad(W1, ((0, H - D_IN), (0, 0))), W2, W3])
    bst = jnp.stack([b1, b2, b3]).reshape(3, 1, H)
    gst = jnp.stack([g1, g2, g3]).reshape(3, 1, H)
    best = jnp.stack([be1, be2, be3]).reshape(3, 1, H)
    rst = jnp.array([0.0, 1.0, 1.0], jnp.float32).reshape(3, 1, 1)
    x0 = jnp.pad(x, ((0, 0), (0, H - D_IN)))

    def _layer(l, xc):
        w = lax.dynamic_index_in_dim(wst, l, 0, keepdims=False)
        brow = lax.dynamic_index_in_dim(bst, l, 0, keepdims=False)
        garow = lax.dynamic_index_in_dim(gst, l, 0, keepdims=False)
        berow = lax.dynamic_index_in_dim(best, l, 0, keepdims=False)
        rs = lax.dynamic_index_in_dim(rst, l, 0, keepdims=False)
        hcat = _mm(xc, w, degcol)
        accf = _sc_agg(hcat, src, dst)
        gpre, s1, s2 = _stats(accf, degcol, brow)
        return _norm(gpre, s1, s2, garow, berow, xc, rs)

    xcur = lax.fori_loop(0, 3, _layer, x0)

    gate, cnt, s1p, slp, gm, mp = _pool1(
        xcur, batchcol, gW1, row(gb1), gW2, row(gb2), lW, row(lb))
    den, zp = _pool2(xcur, gate, batchcol, gm)

    w1p = jnp.pad(hW1, ((0, 1024 - hW1.shape[0]), (0, 0)))
    out = _head(cnt, s1p, mp, zp, den, slp, adme_features,
                w1p, row(hb1), hW2, row(hb2), hW3, row(hb3),
                hW4, row(hb4), hW5, row(hb5))
    return out[:, 0]


# speedup vs baseline: 1.7460x; 1.7460x over previous
import functools

import jax
import jax.numpy as jnp
from jax import lax
from jax.experimental import pallas as pl
from jax.experimental.pallas import tpu as pltpu
from jax.experimental.pallas import tpu_sc as plsc

N = 10000
E = 320000
D_IN = 128
H = 256
B = 128
ADME = 30
NPAD = 10240
HALF = 128
ROWB = 1000
NEG_INF = float("-inf")

EPS_AGG = E // 16
AGG_CHUNKS = EPS_AGG // 128
AGG_REM = EPS_AGG - AGG_CHUNKS * 128
EPS_HIST = E // 32
HIST_CHUNKS = EPS_HIST // 128
HIST_REM = EPS_HIST - HIST_CHUNKS * 128


@functools.lru_cache(maxsize=None)
def _sc_hist_kernel():
    mesh = plsc.VectorSubcoreMesh(core_axis_name="c", subcore_axis_name="s")
    return functools.partial(
        pl.kernel, mesh=mesh,
        out_type=jax.ShapeDtypeStruct((2 * NPAD,), jnp.float32),
        scratch_types=[
            pltpu.VMEM((640,), jnp.float32),
            pltpu.VMEM((128,), jnp.float32),
            pltpu.VMEM((16,), jnp.float32),
            pltpu.VMEM((128,), jnp.int32),
            pltpu.VMEM((16,), jnp.int32),
            pltpu.VMEM_SHARED((NPAD,), jnp.float32),
        ],
    )(_sc_hist_body)


def _sc_hist(dst):
    return _sc_hist_kernel()(dst)


def _sc_hist_body(dst_hbm, out_hbm, zbuf, ones_v, ones16_v, idx_v, idx16_v, acc):
    c = lax.axis_index("c")
    s = lax.axis_index("s")
    wid = s * 2 + c

    def _fill(i, _):
        zbuf[pl.ds(i * 16, 16)] = jnp.zeros((16,), jnp.float32)
        return 0
    lax.fori_loop(0, 40, _fill, 0)
    for k in range(8):
        ones_v[pl.ds(k * 16, 16)] = jnp.ones((16,), jnp.float32)
    ones16_v[...] = jnp.ones((16,), jnp.float32)

    pltpu.sync_copy(zbuf, acc.at[pl.ds(s * 640, 640)])
    plsc.subcore_barrier()

    base = wid * EPS_HIST
    def _chunk(j, _):
        pltpu.sync_copy(dst_hbm.at[pl.ds(base + j * 128, 128)], idx_v)
        pltpu.sync_copy(ones_v, acc.at[idx_v], add=True)
        return 0
    lax.fori_loop(0, HIST_CHUNKS, _chunk, 0)
    pltpu.sync_copy(dst_hbm.at[pl.ds(base + HIST_CHUNKS * 128, 16)], idx16_v)
    pltpu.sync_copy(ones16_v, acc.at[idx16_v], add=True)
    plsc.subcore_barrier()

    pltpu.sync_copy(acc.at[pl.ds(s * 640, 640)], zbuf)
    pltpu.sync_copy(zbuf, out_hbm.at[pl.ds(c * NPAD + s * 640, 640)])


GEDGES = 256
NGROUPS = E // GEDGES
GPT = 78
PR = 5120


@functools.lru_cache(maxsize=None)
def _sc_agg_kernel():
    mesh = plsc.VectorSubcoreMesh(core_axis_name="c", subcore_axis_name="s")
    return functools.partial(
        pl.kernel, mesh=mesh,
        out_type=jax.ShapeDtypeStruct((2 * N, HALF), jnp.float32),
        scratch_types=[
            pltpu.VMEM((64, HALF), jnp.float32),
            pltpu.VMEM((2, 128, HALF), jnp.float32),
            pltpu.VMEM((32, HALF), jnp.float32),
            pltpu.VMEM((128,), jnp.int32),
            pltpu.VMEM((2, 128), jnp.int32),
            pltpu.VMEM((32,), jnp.int32),
            pltpu.VMEM((32,), jnp.int32),
            pltpu.VMEM_SHARED((NPAD, HALF), jnp.float32),
            pltpu.SemaphoreType.DMA,
            pltpu.SemaphoreType.DMA,
        ],
    )(_sc_agg_body)


def _sc_agg(hcat, src, dst):
    return _sc_agg_kernel()(hcat, src, dst)


def _sc_agg_body(hcat_hbm, src_hbm, dst_hbm, out_hbm,
                 stage, msg2, msg32, srcv, didx, srcv32, dstv32, acc,
                 sem_g, sem_s):
    c = lax.axis_index("c")
    s = lax.axis_index("s")
    rbase = s * 640
    gbase = s * GPT
    cn = c * N

    def _mv(off, nrows, into_acc):
        if into_acc:
            pltpu.sync_copy(hcat_hbm.at[pl.ds(cn + off, nrows)],
                            stage.at[pl.ds(0, nrows)])
            pltpu.sync_copy(stage.at[pl.ds(0, nrows)],
                            acc.at[pl.ds(off, nrows)])
        else:
            pltpu.sync_copy(acc.at[pl.ds(off, nrows)],
                            stage.at[pl.ds(0, nrows)])
            pltpu.sync_copy(stage.at[pl.ds(0, nrows)],
                            out_hbm.at[pl.ds(cn + off, nrows)])

    def _copy_stripe(into_acc):
        @pl.when(s < 15)
        def _():
            def _full(k, _):
                _mv(rbase + k * 64, 64, into_acc)
                return 0
            lax.fori_loop(0, 10, _full, 0)

        @pl.when(s == 15)
        def _():
            def _full(k, _):
                _mv(rbase + k * 64, 64, into_acc)
                return 0
            lax.fori_loop(0, 6, _full, 0)
            _mv(rbase + 384, 16, into_acc)

    _copy_stripe(True)
    plsc.subcore_barrier()

    base = s * EPS_AGG

    def _chunk(j, _):
        p = lax.rem(j, 2)
        eo = base + j * 128

        @pl.when(j >= 2)
        def _():
            pltpu.make_async_copy(hcat_hbm.at[pl.ds(0, 128)],
                                  msg2.at[0], sem_s).wait()

        pltpu.sync_copy(src_hbm.at[pl.ds(eo, 128)], srcv)
        pltpu.sync_copy(dst_hbm.at[pl.ds(eo, 128)], didx.at[p])
        for k in range(8):
            srcv[pl.ds(k * 16, 16)] = srcv[pl.ds(k * 16, 16)] + cn
        pltpu.async_copy(hcat_hbm.at[srcv], msg2.at[p], sem_g).wait()
        pltpu.async_copy(msg2.at[p], acc.at[didx.at[p]], sem_s, add=True)
        return 0
    lax.fori_loop(0, AGG_CHUNKS, _chunk, 0)

    pltpu.make_async_copy(hcat_hbm.at[pl.ds(0, 128)], msg2.at[0], sem_s).wait()
    pltpu.make_async_copy(hcat_hbm.at[pl.ds(0, 128)], msg2.at[0], sem_s).wait()

    eo = base + AGG_CHUNKS * 128
    pltpu.sync_copy(src_hbm.at[pl.ds(eo, AGG_REM)], srcv32)
    pltpu.sync_copy(dst_hbm.at[pl.ds(eo, AGG_REM)], dstv32)
    for k in range(AGG_REM // 16):
        srcv32[pl.ds(k * 16, 16)] = srcv32[pl.ds(k * 16, 16)] + cn
    pltpu.async_copy(hcat_hbm.at[srcv32], msg32, sem_g).wait()
    pltpu.sync_copy(msg32, acc.at[dstv32], add=True)

    plsc.subcore_barrier()
    _copy_stripe(False)


def _mm_body(x_ref, w_ref, deg_ref, out_ref):
    dinv = lax.rsqrt(deg_ref[...])
    out_ref[...] = jnp.dot(x_ref[...], w_ref[...],
                           preferred_element_type=jnp.float32) * dinv


def _mm(xin, w, degcol):
    k = xin.shape[1]
    return pl.pallas_call(
        _mm_body,
        grid=(20,),
        in_specs=[
            pl.BlockSpec((ROWB, k), lambda i: (i % 10, 0)),
            pl.BlockSpec((k, HALF), lambda i: (0, i // 10)),
            pl.BlockSpec((ROWB, 1), lambda i: (i % 10, 0)),
        ],
        out_specs=pl.BlockSpec((ROWB, HALF), lambda i: (i, 0)),
        out_shape=jax.ShapeDtypeStruct((2 * N, HALF), jnp.float32),
    )(xin, w, degcol)


def _stats_body(a0_ref, a1_ref, deg_ref, b_ref, gpre_ref, s1_ref, s2_ref):
    dinv = lax.rsqrt(deg_ref[...])
    g = jnp.concatenate([a0_ref[...], a1_ref[...]], axis=1) * dinv + b_ref[...]
    gpre_ref[...] = g

    @pl.when(pl.program_id(0) == 0)
    def _():
        s1_ref[...] = jnp.zeros_like(s1_ref)
        s2_ref[...] = jnp.zeros_like(s2_ref)

    s1_ref[...] += jnp.sum(g, axis=0, keepdims=True)
    s2_ref[...] += jnp.sum(g * g, axis=0, keepdims=True)


def _stats(accf, degcol, brow):
    return pl.pallas_call(
        _stats_body,
        grid=(10,),
        in_specs=[
            pl.BlockSpec((ROWB, HALF), lambda i: (i, 0)),
            pl.BlockSpec((ROWB, HALF), lambda i: (i + 10, 0)),
            pl.BlockSpec((ROWB, 1), lambda i: (i, 0)),
            pl.BlockSpec((1, H), lambda i: (0, 0)),
        ],
        out_specs=[
            pl.BlockSpec((ROWB, H), lambda i: (i, 0)),
            pl.BlockSpec((1, H), lambda i: (0, 0)),
            pl.BlockSpec((1, H), lambda i: (0, 0)),
        ],
        out_shape=[
            jax.ShapeDtypeStruct((N, H), jnp.float32),
            jax.ShapeDtypeStruct((1, H), jnp.float32),
            jax.ShapeDtypeStruct((1, H), jnp.float32),
        ],
    )(accf, accf, degcol, brow)


def _norm_body(g_ref, s1_ref, s2_ref, ga_ref, be_ref, prev_ref, rs_ref,
               out_ref):
    m = s1_ref[...] * (1.0 / N)
    v = s2_ref[...] * (1.0 / N) - m * m
    rstd = lax.rsqrt(v + 1e-5)
    y = (g_ref[...] - m) * rstd * ga_ref[...] + be_ref[...]
    y = jnp.maximum(y, 0.0)
    out_ref[...] = y + prev_ref[...] * rs_ref[...]


def _norm(gpre, s1, s2, garow, berow, xprev, rscale):
    return pl.pallas_call(
        _norm_body,
        grid=(10,),
        in_specs=[
            pl.BlockSpec((ROWB, H), lambda i: (i, 0)),
            pl.BlockSpec((1, H), lambda i: (0, 0)),
            pl.BlockSpec((1, H), lambda i: (0, 0)),
            pl.BlockSpec((1, H), lambda i: (0, 0)),
            pl.BlockSpec((1, H), lambda i: (0, 0)),
            pl.BlockSpec((ROWB, H), lambda i: (i, 0)),
            pl.BlockSpec((1, 1), lambda i: (0, 0)),
        ],
        out_specs=pl.BlockSpec((ROWB, H), lambda i: (i, 0)),
        out_shape=jax.ShapeDtypeStruct((N, H), jnp.float32),
    )(gpre, s1, s2, garow, berow, xprev, rscale)


def _gelu(x):
    return 0.5 * x * (1.0 + lax.erf(x * 0.7071067811865476))


def _pool1_body(x3_ref, b_ref, gw1_ref, gb1_ref, gw2_ref, gb2_ref,
                lw_ref, lb_ref,
                gate_ref, cnt_ref, s1_ref, sl_ref, gm_ref, m_ref):
    x3 = x3_ref[...]
    t = _gelu(jnp.dot(x3, gw1_ref[...],
                      preferred_element_type=jnp.float32) + gb1_ref[...])
    gate = jnp.dot(t, gw2_ref[...],
                   preferred_element_type=jnp.float32) + gb2_ref[...]
    gate_ref[...] = gate
    loc = _gelu(jnp.dot(x3, lw_ref[...],
                        preferred_element_type=jnp.float32) + lb_ref[...])

    bcol = b_ref[...]
    io = lax.broadcasted_iota(jnp.int32, (ROWB, B), 1)
    ob = bcol == io
    ohf = ob.astype(jnp.float32)
    ones_col = jnp.ones((ROWB, 1), jnp.float32)
    dn = (((0,), (0,)), ((), ()))
    cntc = lax.dot_general(ohf, ones_col, dn,
                           preferred_element_type=jnp.float32)
    s1c = lax.dot_general(ohf, x3, dn,
                          preferred_element_type=jnp.float32)
    slc = lax.dot_general(ohf, loc, dn,
                          preferred_element_type=jnp.float32)
    gmc = jnp.max(jnp.where(ob, gate, NEG_INF), axis=0, keepdims=True)

    @pl.when(pl.program_id(0) == 0)
    def _():
        cnt_ref[...] = jnp.zeros_like(cnt_ref)
        s1_ref[...] = jnp.zeros_like(s1_ref)
        sl_ref[...] = jnp.zeros_like(sl_ref)
        gm_ref[...] = jnp.full_like(gm_ref, NEG_INF)
        m_ref[...] = jnp.full_like(m_ref, NEG_INF)

    cnt_ref[...] += cntc
    s1_ref[...] += s1c
    sl_ref[...] += slc
    gm_ref[...] = jnp.maximum(gm_ref[...], gmc)

    bmin = jnp.min(bcol)
    bmax = jnp.max(bcol)
    rio = lax.broadcasted_iota(jnp.int32, (B, 1), 0)

    def _mb(bi, _):
        mask = bcol == bi
        mrow = jnp.max(jnp.where(mask, x3, NEG_INF), axis=0, keepdims=True)
        cur = m_ref[...]
        m_ref[...] = jnp.where(rio == bi, jnp.maximum(cur, mrow), cur)
        return 0

    lax.fori_loop(bmin, bmax + 1, _mb, 0)


def _pool1(x3, batchcol, gw1, gb1, gw2, gb2, lw, lb):
    return pl.pallas_call(
        _pool1_body,
        grid=(10,),
        in_specs=[
            pl.BlockSpec((ROWB, H), lambda i: (i, 0)),
            pl.BlockSpec((ROWB, 1), lambda i: (i, 0)),
            pl.BlockSpec((H, 128), lambda i: (0, 0)),
            pl.BlockSpec((1, 128), lambda i: (0, 0)),
            pl.BlockSpec((128, 1), lambda i: (0, 0)),
            pl.BlockSpec((1, 1), lambda i: (0, 0)),
            pl.BlockSpec((H, 128), lambda i: (0, 0)),
            pl.BlockSpec((1, 128), lambda i: (0, 0)),
        ],
        out_specs=[
            pl.BlockSpec((ROWB, 1), lambda i: (i, 0)),
            pl.BlockSpec((B, 1), lambda i: (0, 0)),
            pl.BlockSpec((B, H), lambda i: (0, 0)),
            pl.BlockSpec((B, 128), lambda i: (0, 0)),
            pl.BlockSpec((1, B), lambda i: (0, 0)),
            pl.BlockSpec((B, H), lambda i: (0, 0)),
        ],
        out_shape=[
            jax.ShapeDtypeStruct((N, 1), jnp.float32),
            jax.ShapeDtypeStruct((B, 1), jnp.float32),
            jax.ShapeDtypeStruct((B, H), jnp.float32),
            jax.ShapeDtypeStruct((B, 128), jnp.float32),
            jax.ShapeDtypeStruct((1, B), jnp.float32),
            jax.ShapeDtypeStruct((B, H), jnp.float32),
        ],
    )(x3, batchcol, gw1, gb1, gw2, gb2, lw, lb)


def _pool2_body(x3_ref, gate_ref, b_ref, gm_ref, den_ref, z_ref):
    x3 = x3_ref[...]
    gate = gate_ref[...]
    bcol = b_ref[...]
    io = lax.broadcasted_iota(jnp.int32, (ROWB, B), 1)
    ob = bcol == io
    ohf = ob.astype(jnp.float32)
    gmb = jnp.sum(jnp.where(ob, gm_ref[...], 0.0), axis=1, keepdims=True)
    e = jnp.exp(gate - gmb)
    dn = (((0,), (0,)), ((), ()))
    denc = lax.dot_general(ohf, e, dn,
                           preferred_element_type=jnp.float32)
    zc = lax.dot_general(ohf * e, x3, dn,
                         preferred_element_type=jnp.float32)

    @pl.when(pl.program_id(0) == 0)
    def _():
        den_ref[...] = jnp.zeros_like(den_ref)
        z_ref[...] = jnp.zeros_like(z_ref)

    den_ref[...] += denc
    z_ref[...] += zc


def _pool2(x3, gate, batchcol, gm):
    return pl.pallas_call(
        _pool2_body,
        grid=(10,),
        in_specs=[
            pl.BlockSpec((ROWB, H), lambda i: (i, 0)),
            pl.BlockSpec((ROWB, 1), lambda i: (i, 0)),
            pl.BlockSpec((ROWB, 1), lambda i: (i, 0)),
            pl.BlockSpec((1, B), lambda i: (0, 0)),
        ],
        out_specs=[
            pl.BlockSpec((B, 1), lambda i: (0, 0)),
            pl.BlockSpec((B, H), lambda i: (0, 0)),
        ],
        out_shape=[
            jax.ShapeDtypeStruct((B, 1), jnp.float32),
            jax.ShapeDtypeStruct((B, H), jnp.float32),
        ],
    )(x3, gate, batchcol, gm)


def _head_body(cnt_ref, s1_ref, m_ref, z_ref, den_ref, sl_ref, adme_ref,
               w1_ref, b1_ref, w2_ref, b2_ref, w3_ref, b3_ref,
               w4_ref, b4_ref, w5_ref, b5_ref, out_ref, comb):
    c = jnp.maximum(cnt_ref[...], 1.0)
    comb[:, 0:256] = s1_ref[...] / c
    comb[:, 256:512] = m_ref[...]
    comb[:, 512:768] = z_ref[...] / den_ref[...]
    comb[:, 768:896] = sl_ref[...] / c
    comb[:, 896:1024] = jnp.concatenate(
        [adme_ref[...], jnp.zeros((B, 98), jnp.float32)], axis=1)
    h = comb[...]
    h = jnp.maximum(jnp.dot(h, w1_ref[...],
                            preferred_element_type=jnp.float32)
                    + b1_ref[...], 0.0)
    h = jnp.maximum(jnp.dot(h, w2_ref[...],
                            preferred_element_type=jnp.float32)
                    + b2_ref[...], 0.0)
    h = jnp.maximum(jnp.dot(h, w3_ref[...],
                            preferred_element_type=jnp.float32)
                    + b3_ref[...], 0.0)
    h = jnp.maximum(jnp.dot(h, w4_ref[...],
                            preferred_element_type=jnp.float32)
                    + b4_ref[...], 0.0)
    out_ref[...] = jnp.dot(h, w5_ref[...],
                           preferred_element_type=jnp.float32) + b5_ref[...]


def _head(cnt, s1, m, z, den, sl, adme, w1p, b1, w2, b2, w3, b3, w4, b4,
          w5, b5):
    return pl.pallas_call(
        _head_body,
        out_shape=jax.ShapeDtypeStruct((B, 1), jnp.float32),
        scratch_shapes=[pltpu.VMEM((B, 1024), jnp.float32)],
    )(cnt, s1, m, z, den, sl, adme, w1p, b1, w2, b2, w3, b3, w4, b4, w5, b5)


def kernel(x, edge_index, batch, adme_features, W1, b1, W2, b2, W3, b3,
           g1, be1, g2, be2, g3, be3, gW1, gb1, gW2, gb2, lW, lb,
           hW1, hb1, hW2, hb2, hW3, hb3, hW4, hb4, hW5, hb5):
    src = edge_index[0]
    dst = edge_index[1]

    hist = _sc_hist(dst)
    degcol = (hist[:NPAD][:N] + hist[NPAD:][:N] + 1.0).reshape(N, 1)

    batchcol = batch.reshape(N, 1)
    row = lambda v: v.reshape(1, -1)

    wst = jnp.stack([jnp.pad(W1, ((0, H - D_IN), (0, 0))), W2, W3])
    bst = jnp.stack([b1, b2, b3]).reshape(3, 1, H)
    gst = jnp.stack([g1, g2, g3]).reshape(3, 1, H)
    best = jnp.stack([be1, be2, be3]).reshape(3, 1, H)
    rst = jnp.array([0.0, 1.0, 1.0], jnp.float32).reshape(3, 1, 1)
    x0 = jnp.pad(x, ((0, 0), (0, H - D_IN)))

    def _layer(l, xc):
        w = wst[l]
        brow = bst[l]
        garow = gst[l]
        berow = best[l]
        rs = rst[l]
        hcat = _mm(xc, w, degcol)
        accf = _sc_agg(hcat, src, dst)
        gpre, s1, s2 = _stats(accf, degcol, brow)
        return _norm(gpre, s1, s2, garow, berow, xc, rs)

    xcur = x0
    for l in range(3):
        xcur = _layer(l, xcur)

    gate, cnt, s1p, slp, gm, mp = _pool1(
        xcur, batchcol, gW1, row(gb1), gW2, row(gb2), lW, row(lb))
    den, zp = _pool2(xcur, gate, batchcol, gm)

    w1p = jnp.pad(hW1, ((0, 1024 - hW1.shape[0]), (0, 0)))
    out = _head(cnt, s1p, mp, zp, den, slp, adme_features,
                w1p, row(hb1), hW2, row(hb2), hW3, row(hb3),
                hW4, row(hb4), hW5, row(hb5))
    return out[:, 0]


# speedup vs baseline: 2.0099x; 1.1511x over previous
import functools

import jax
import jax.numpy as jnp
from jax import lax
from jax.experimental import pallas as pl
from jax.experimental.pallas import tpu as pltpu
from jax.experimental.pallas import tpu_sc as plsc

N = 10000
E = 320000
D_IN = 128
H = 256
B = 128
ADME = 30
NPAD = 10240
HALF = 128
ROWB = 1000
NEG_INF = float("-inf")

EPS_AGG = E // 16
AGG_CHUNKS = EPS_AGG // 128
AGG_REM = EPS_AGG - AGG_CHUNKS * 128
EPS_HIST = E // 32
HIST_CHUNKS = EPS_HIST // 128
HIST_REM = EPS_HIST - HIST_CHUNKS * 128


@functools.lru_cache(maxsize=None)
def _sc_hist_kernel():
    mesh = plsc.VectorSubcoreMesh(core_axis_name="c", subcore_axis_name="s")
    return functools.partial(
        pl.kernel, mesh=mesh,
        out_type=jax.ShapeDtypeStruct((2 * NPAD,), jnp.float32),
        scratch_types=[
            pltpu.VMEM((640,), jnp.float32),
            pltpu.VMEM((128,), jnp.float32),
            pltpu.VMEM((16,), jnp.float32),
            pltpu.VMEM((128,), jnp.int32),
            pltpu.VMEM((16,), jnp.int32),
            pltpu.VMEM_SHARED((NPAD,), jnp.float32),
        ],
    )(_sc_hist_body)


def _sc_hist(dst):
    return _sc_hist_kernel()(dst)


def _sc_hist_body(dst_hbm, out_hbm, zbuf, ones_v, ones16_v, idx_v, idx16_v, acc):
    c = lax.axis_index("c")
    s = lax.axis_index("s")
    wid = s * 2 + c

    def _fill(i, _):
        zbuf[pl.ds(i * 16, 16)] = jnp.zeros((16,), jnp.float32)
        return 0
    lax.fori_loop(0, 40, _fill, 0)
    for k in range(8):
        ones_v[pl.ds(k * 16, 16)] = jnp.ones((16,), jnp.float32)
    ones16_v[...] = jnp.ones((16,), jnp.float32)

    pltpu.sync_copy(zbuf, acc.at[pl.ds(s * 640, 640)])
    plsc.subcore_barrier()

    base = wid * EPS_HIST
    def _chunk(j, _):
        pltpu.sync_copy(dst_hbm.at[pl.ds(base + j * 128, 128)], idx_v)
        pltpu.sync_copy(ones_v, acc.at[idx_v], add=True)
        return 0
    lax.fori_loop(0, HIST_CHUNKS, _chunk, 0)
    pltpu.sync_copy(dst_hbm.at[pl.ds(base + HIST_CHUNKS * 128, 16)], idx16_v)
    pltpu.sync_copy(ones16_v, acc.at[idx16_v], add=True)
    plsc.subcore_barrier()

    pltpu.sync_copy(acc.at[pl.ds(s * 640, 640)], zbuf)
    pltpu.sync_copy(zbuf, out_hbm.at[pl.ds(c * NPAD + s * 640, 640)])


GEDGES = 256
NGROUPS = E // GEDGES
GPT = 78
PR = 5120


@functools.lru_cache(maxsize=None)
def _sc_agg_kernel():
    mesh = plsc.VectorSubcoreMesh(core_axis_name="c", subcore_axis_name="s")
    return functools.partial(
        pl.kernel, mesh=mesh,
        out_type=jax.ShapeDtypeStruct((2 * N, HALF), jnp.float32),
        scratch_types=[
            pltpu.VMEM((64, HALF), jnp.float32),
            pltpu.VMEM((2, 128, HALF), jnp.float32),
            pltpu.VMEM((2, 2, 128), jnp.int32),
            pltpu.VMEM_SHARED((NPAD, HALF), jnp.float32),
            pltpu.SemaphoreType.DMA,
            pltpu.SemaphoreType.DMA,
        ],
    )(_sc_agg_body)


def _sc_agg(hcat, edges3):
    return _sc_agg_kernel()(hcat, edges3)


def _sc_agg_body(hcat_hbm, edges_hbm, out_hbm,
                 stage, msg2, ed2, acc, sem_g, sem_s):
    c = lax.axis_index("c")
    s = lax.axis_index("s")
    rbase = s * 640
    gbase = s * GPT
    cn = c * N

    def _mv(off, nrows, into_acc):
        if into_acc:
            pltpu.sync_copy(hcat_hbm.at[pl.ds(cn + off, nrows)],
                            stage.at[pl.ds(0, nrows)])
            pltpu.sync_copy(stage.at[pl.ds(0, nrows)],
                            acc.at[pl.ds(off, nrows)])
        else:
            pltpu.sync_copy(acc.at[pl.ds(off, nrows)],
                            stage.at[pl.ds(0, nrows)])
            pltpu.sync_copy(stage.at[pl.ds(0, nrows)],
                            out_hbm.at[pl.ds(cn + off, nrows)])

    def _copy_stripe(into_acc):
        @pl.when(s < 15)
        def _():
            def _full(k, _):
                _mv(rbase + k * 64, 64, into_acc)
                return 0
            lax.fori_loop(0, 10, _full, 0)

        @pl.when(s == 15)
        def _():
            def _full(k, _):
                _mv(rbase + k * 64, 64, into_acc)
                return 0
            lax.fori_loop(0, 6, _full, 0)
            _mv(rbase + 384, 16, into_acc)

    _copy_stripe(True)
    plsc.subcore_barrier()

    nchunks = jnp.where(s < 4, 157, 156)

    def _chunk(j, _):
        p = lax.rem(j, 2)
        row = c * 2500 + jnp.where(j < 156, s * 156 + j, 2496 + s)

        @pl.when(j >= 2)
        def _():
            pltpu.make_async_copy(hcat_hbm.at[pl.ds(0, 128)],
                                  msg2.at[0], sem_s).wait()

        pltpu.sync_copy(edges_hbm.at[row], ed2.at[p])
        pltpu.async_copy(hcat_hbm.at[ed2.at[p, 0]], msg2.at[p], sem_g).wait()
        pltpu.async_copy(msg2.at[p], acc.at[ed2.at[p, 1]], sem_s, add=True)
        return 0
    lax.fori_loop(0, nchunks, _chunk, 0)

    pltpu.make_async_copy(hcat_hbm.at[pl.ds(0, 128)], msg2.at[0], sem_s).wait()
    pltpu.make_async_copy(hcat_hbm.at[pl.ds(0, 128)], msg2.at[0], sem_s).wait()

    plsc.subcore_barrier()
    _copy_stripe(False)


def _edges_body(s_ref, d_ref, out_ref):
    cc = pl.program_id(0)
    out_ref[...] = jnp.stack([s_ref[...] + cc * N, d_ref[...]], axis=1)


def _edges_prep(src2d, dst2d):
    nr = E // 128
    return pl.pallas_call(
        _edges_body,
        grid=(2,),
        in_specs=[
            pl.BlockSpec((nr, 128), lambda i: (0, 0)),
            pl.BlockSpec((nr, 128), lambda i: (0, 0)),
        ],
        out_specs=pl.BlockSpec((nr, 2, 128), lambda i: (i, 0, 0)),
        out_shape=jax.ShapeDtypeStruct((2 * nr, 2, 128), jnp.int32),
    )(src2d, dst2d)


def _mm_body(x_ref, w_ref, deg_ref, out_ref):
    dinv = lax.rsqrt(deg_ref[...])
    out_ref[...] = jnp.dot(x_ref[...], w_ref[...],
                           preferred_element_type=jnp.float32) * dinv


def _mm(xin, w, degcol):
    k = xin.shape[1]
    return pl.pallas_call(
        _mm_body,
        grid=(20,),
        in_specs=[
            pl.BlockSpec((ROWB, k), lambda i: (i % 10, 0)),
            pl.BlockSpec((k, HALF), lambda i: (0, i // 10)),
            pl.BlockSpec((ROWB, 1), lambda i: (i % 10, 0)),
        ],
        out_specs=pl.BlockSpec((ROWB, HALF), lambda i: (i, 0)),
        out_shape=jax.ShapeDtypeStruct((2 * N, HALF), jnp.float32),
    )(xin, w, degcol)


def _stats_body(a0_ref, a1_ref, deg_ref, b_ref, gpre_ref, s1_ref, s2_ref):
    dinv = lax.rsqrt(deg_ref[...])
    g = jnp.concatenate([a0_ref[...], a1_ref[...]], axis=1) * dinv + b_ref[...]
    gpre_ref[...] = g

    @pl.when(pl.program_id(0) == 0)
    def _():
        s1_ref[...] = jnp.zeros_like(s1_ref)
        s2_ref[...] = jnp.zeros_like(s2_ref)

    s1_ref[...] += jnp.sum(g, axis=0, keepdims=True)
    s2_ref[...] += jnp.sum(g * g, axis=0, keepdims=True)


def _stats(accf, degcol, brow):
    return pl.pallas_call(
        _stats_body,
        grid=(10,),
        in_specs=[
            pl.BlockSpec((ROWB, HALF), lambda i: (i, 0)),
            pl.BlockSpec((ROWB, HALF), lambda i: (i + 10, 0)),
            pl.BlockSpec((ROWB, 1), lambda i: (i, 0)),
            pl.BlockSpec((1, H), lambda i: (0, 0)),
        ],
        out_specs=[
            pl.BlockSpec((ROWB, H), lambda i: (i, 0)),
            pl.BlockSpec((1, H), lambda i: (0, 0)),
            pl.BlockSpec((1, H), lambda i: (0, 0)),
        ],
        out_shape=[
            jax.ShapeDtypeStruct((N, H), jnp.float32),
            jax.ShapeDtypeStruct((1, H), jnp.float32),
            jax.ShapeDtypeStruct((1, H), jnp.float32),
        ],
    )(accf, accf, degcol, brow)


def _norm_body(g_ref, s1_ref, s2_ref, ga_ref, be_ref, prev_ref, rs_ref,
               out_ref):
    m = s1_ref[...] * (1.0 / N)
    v = s2_ref[...] * (1.0 / N) - m * m
    rstd = lax.rsqrt(v + 1e-5)
    y = (g_ref[...] - m) * rstd * ga_ref[...] + be_ref[...]
    y = jnp.maximum(y, 0.0)
    out_ref[...] = y + prev_ref[...] * rs_ref[...]


def _norm(gpre, s1, s2, garow, berow, xprev, rscale):
    return pl.pallas_call(
        _norm_body,
        grid=(10,),
        in_specs=[
            pl.BlockSpec((ROWB, H), lambda i: (i, 0)),
            pl.BlockSpec((1, H), lambda i: (0, 0)),
            pl.BlockSpec((1, H), lambda i: (0, 0)),
            pl.BlockSpec((1, H), lambda i: (0, 0)),
            pl.BlockSpec((1, H), lambda i: (0, 0)),
            pl.BlockSpec((ROWB, H), lambda i: (i, 0)),
            pl.BlockSpec((1, 1), lambda i: (0, 0)),
        ],
        out_specs=pl.BlockSpec((ROWB, H), lambda i: (i, 0)),
        out_shape=jax.ShapeDtypeStruct((N, H), jnp.float32),
    )(gpre, s1, s2, garow, berow, xprev, rscale)


def _gelu(x):
    return 0.5 * x * (1.0 + lax.erf(x * 0.7071067811865476))


def _pool1_body(x3_ref, b_ref, gw1_ref, gb1_ref, gw2_ref, gb2_ref,
                lw_ref, lb_ref,
                gate_ref, cnt_ref, s1_ref, sl_ref, gm_ref, m_ref):
    x3 = x3_ref[...]
    t = _gelu(jnp.dot(x3, gw1_ref[...],
                      preferred_element_type=jnp.float32) + gb1_ref[...])
    gate = jnp.dot(t, gw2_ref[...],
                   preferred_element_type=jnp.float32) + gb2_ref[...]
    gate_ref[...] = gate
    loc = _gelu(jnp.dot(x3, lw_ref[...],
                        preferred_element_type=jnp.float32) + lb_ref[...])

    bcol = b_ref[...]
    io = lax.broadcasted_iota(jnp.int32, (ROWB, B), 1)
    ob = bcol == io
    ohf = ob.astype(jnp.float32)
    ones_col = jnp.ones((ROWB, 1), jnp.float32)
    dn = (((0,), (0,)), ((), ()))
    cntc = lax.dot_general(ohf, ones_col, dn,
                           preferred_element_type=jnp.float32)
    s1c = lax.dot_general(ohf, x3, dn,
                          preferred_element_type=jnp.float32)
    slc = lax.dot_general(ohf, loc, dn,
                          preferred_element_type=jnp.float32)
    gmc = jnp.max(jnp.where(ob, gate, NEG_INF), axis=0, keepdims=True)

    @pl.when(pl.program_id(0) == 0)
    def _():
        cnt_ref[...] = jnp.zeros_like(cnt_ref)
        s1_ref[...] = jnp.zeros_like(s1_ref)
        sl_ref[...] = jnp.zeros_like(sl_ref)
        gm_ref[...] = jnp.full_like(gm_ref, NEG_INF)
        m_ref[...] = jnp.full_like(m_ref, NEG_INF)

    cnt_ref[...] += cntc
    s1_ref[...] += s1c
    sl_ref[...] += slc
    gm_ref[...] = jnp.maximum(gm_ref[...], gmc)

    bmin = jnp.min(bcol)
    bmax = jnp.max(bcol)
    rio = lax.broadcasted_iota(jnp.int32, (B, 1), 0)

    def _mb(bi, _):
        mask = bcol == bi
        mrow = jnp.max(jnp.where(mask, x3, NEG_INF), axis=0, keepdims=True)
        cur = m_ref[...]
        m_ref[...] = jnp.where(rio == bi, jnp.maximum(cur, mrow), cur)
        return 0

    lax.fori_loop(bmin, bmax + 1, _mb, 0)


def _pool1(x3, batchcol, gw1, gb1, gw2, gb2, lw, lb):
    return pl.pallas_call(
        _pool1_body,
        grid=(10,),
        in_specs=[
            pl.BlockSpec((ROWB, H), lambda i: (i, 0)),
            pl.BlockSpec((ROWB, 1), lambda i: (i, 0)),
            pl.BlockSpec((H, 128), lambda i: (0, 0)),
            pl.BlockSpec((1, 128), lambda i: (0, 0)),
            pl.BlockSpec((128, 1), lambda i: (0, 0)),
            pl.BlockSpec((1, 1), lambda i: (0, 0)),
            pl.BlockSpec((H, 128), lambda i: (0, 0)),
            pl.BlockSpec((1, 128), lambda i: (0, 0)),
        ],
        out_specs=[
            pl.BlockSpec((ROWB, 1), lambda i: (i, 0)),
            pl.BlockSpec((B, 1), lambda i: (0, 0)),
            pl.BlockSpec((B, H), lambda i: (0, 0)),
            pl.BlockSpec((B, 128), lambda i: (0, 0)),
            pl.BlockSpec((1, B), lambda i: (0, 0)),
            pl.BlockSpec((B, H), lambda i: (0, 0)),
        ],
        out_shape=[
            jax.ShapeDtypeStruct((N, 1), jnp.float32),
            jax.ShapeDtypeStruct((B, 1), jnp.float32),
            jax.ShapeDtypeStruct((B, H), jnp.float32),
            jax.ShapeDtypeStruct((B, 128), jnp.float32),
            jax.ShapeDtypeStruct((1, B), jnp.float32),
            jax.ShapeDtypeStruct((B, H), jnp.float32),
        ],
    )(x3, batchcol, gw1, gb1, gw2, gb2, lw, lb)


def _pool2_body(x3_ref, gate_ref, b_ref, gm_ref, den_ref, z_ref):
    x3 = x3_ref[...]
    gate = gate_ref[...]
    bcol = b_ref[...]
    io = lax.broadcasted_iota(jnp.int32, (ROWB, B), 1)
    ob = bcol == io
    ohf = ob.astype(jnp.float32)
    gmb = jnp.sum(jnp.where(ob, gm_ref[...], 0.0), axis=1, keepdims=True)
    e = jnp.exp(gate - gmb)
    dn = (((0,), (0,)), ((), ()))
    denc = lax.dot_general(ohf, e, dn,
                           preferred_element_type=jnp.float32)
    zc = lax.dot_general(ohf * e, x3, dn,
                         preferred_element_type=jnp.float32)

    @pl.when(pl.program_id(0) == 0)
    def _():
        den_ref[...] = jnp.zeros_like(den_ref)
        z_ref[...] = jnp.zeros_like(z_ref)

    den_ref[...] += denc
    z_ref[...] += zc


def _pool2(x3, gate, batchcol, gm):
    return pl.pallas_call(
        _pool2_body,
        grid=(10,),
        in_specs=[
            pl.BlockSpec((ROWB, H), lambda i: (i, 0)),
            pl.BlockSpec((ROWB, 1), lambda i: (i, 0)),
            pl.BlockSpec((ROWB, 1), lambda i: (i, 0)),
            pl.BlockSpec((1, B), lambda i: (0, 0)),
        ],
        out_specs=[
            pl.BlockSpec((B, 1), lambda i: (0, 0)),
            pl.BlockSpec((B, H), lambda i: (0, 0)),
        ],
        out_shape=[
            jax.ShapeDtypeStruct((B, 1), jnp.float32),
            jax.ShapeDtypeStruct((B, H), jnp.float32),
        ],
    )(x3, gate, batchcol, gm)


def _head_body(cnt_ref, s1_ref, m_ref, z_ref, den_ref, sl_ref, adme_ref,
               w1_ref, b1_ref, w2_ref, b2_ref, w3_ref, b3_ref,
               w4_ref, b4_ref, w5_ref, b5_ref, out_ref, comb):
    c = jnp.maximum(cnt_ref[...], 1.0)
    comb[:, 0:256] = s1_ref[...] / c
    comb[:, 256:512] = m_ref[...]
    comb[:, 512:768] = z_ref[...] / den_ref[...]
    comb[:, 768:896] = sl_ref[...] / c
    comb[:, 896:1024] = jnp.concatenate(
        [adme_ref[...], jnp.zeros((B, 98), jnp.float32)], axis=1)
    h = comb[...]
    h = jnp.maximum(jnp.dot(h, w1_ref[...],
                            preferred_element_type=jnp.float32)
                    + b1_ref[...], 0.0)
    h = jnp.maximum(jnp.dot(h, w2_ref[...],
                            preferred_element_type=jnp.float32)
                    + b2_ref[...], 0.0)
    h = jnp.maximum(jnp.dot(h, w3_ref[...],
                            preferred_element_type=jnp.float32)
                    + b3_ref[...], 0.0)
    h = jnp.maximum(jnp.dot(h, w4_ref[...],
                            preferred_element_type=jnp.float32)
                    + b4_ref[...], 0.0)
    out_ref[...] = jnp.dot(h, w5_ref[...],
                           preferred_element_type=jnp.float32) + b5_ref[...]


def _head(cnt, s1, m, z, den, sl, adme, w1p, b1, w2, b2, w3, b3, w4, b4,
          w5, b5):
    return pl.pallas_call(
        _head_body,
        out_shape=jax.ShapeDtypeStruct((B, 1), jnp.float32),
        scratch_shapes=[pltpu.VMEM((B, 1024), jnp.float32)],
    )(cnt, s1, m, z, den, sl, adme, w1p, b1, w2, b2, w3, b3, w4, b4, w5, b5)


def kernel(x, edge_index, batch, adme_features, W1, b1, W2, b2, W3, b3,
           g1, be1, g2, be2, g3, be3, gW1, gb1, gW2, gb2, lW, lb,
           hW1, hb1, hW2, hb2, hW3, hb3, hW4, hb4, hW5, hb5):
    src = edge_index[0]
    dst = edge_index[1]
    edges3 = _edges_prep(src.reshape(E // 128, 128),
                         dst.reshape(E // 128, 128))

    hist = _sc_hist(dst)
    degcol = (hist[:NPAD][:N] + hist[NPAD:][:N] + 1.0).reshape(N, 1)

    batchcol = batch.reshape(N, 1)
    row = lambda v: v.reshape(1, -1)

    wst = jnp.stack([jnp.pad(W1, ((0, H - D_IN), (0, 0))), W2, W3])
    bst = jnp.stack([b1, b2, b3]).reshape(3, 1, H)
    gst = jnp.stack([g1, g2, g3]).reshape(3, 1, H)
    best = jnp.stack([be1, be2, be3]).reshape(3, 1, H)
    rst = jnp.array([0.0, 1.0, 1.0], jnp.float32).reshape(3, 1, 1)
    x0 = jnp.pad(x, ((0, 0), (0, H - D_IN)))

    def _layer(l, xc):
        w = wst[l]
        brow = bst[l]
        garow = gst[l]
        berow = best[l]
        rs = rst[l]
        hcat = _mm(xc, w, degcol)
        accf = _sc_agg(hcat, edges3)
        gpre, s1, s2 = _stats(accf, degcol, brow)
        return _norm(gpre, s1, s2, garow, berow, xc, rs)

    xcur = x0
    for l in range(3):
        xcur = _layer(l, xcur)

    gate, cnt, s1p, slp, gm, mp = _pool1(
        xcur, batchcol, gW1, row(gb1), gW2, row(gb2), lW, row(lb))
    den, zp = _pool2(xcur, gate, batchcol, gm)

    w1p = jnp.pad(hW1, ((0, 1024 - hW1.shape[0]), (0, 0)))
    out = _head(cnt, s1p, mp, zp, den, slp, adme_features,
                w1p, row(hb1), hW2, row(hb2), hW3, row(hb3),
                hW4, row(hb4), hW5, row(hb5))
    return out[:, 0]


# speedup vs baseline: 2.4212x; 1.2046x over previous
import functools

import jax
import jax.numpy as jnp
from jax import lax
from jax.experimental import pallas as pl
from jax.experimental.pallas import tpu as pltpu
from jax.experimental.pallas import tpu_sc as plsc

N = 10000
E = 320000
D_IN = 128
H = 256
B = 128
ADME = 30
NPAD = 10240
HALF = 128
ROWB = 1000
NEG_INF = float("-inf")

EPS_AGG = E // 16
AGG_CHUNKS = EPS_AGG // 128
AGG_REM = EPS_AGG - AGG_CHUNKS * 128
EPS_HIST = E // 32
HIST_CHUNKS = EPS_HIST // 128
HIST_REM = EPS_HIST - HIST_CHUNKS * 128


@functools.lru_cache(maxsize=None)
def _sc_hist_kernel():
    mesh = plsc.VectorSubcoreMesh(core_axis_name="c", subcore_axis_name="s")
    return functools.partial(
        pl.kernel, mesh=mesh,
        out_type=jax.ShapeDtypeStruct((2 * NPAD,), jnp.float32),
        scratch_types=[
            pltpu.VMEM((640,), jnp.float32),
            pltpu.VMEM((128,), jnp.float32),
            pltpu.VMEM((16,), jnp.float32),
            pltpu.VMEM((128,), jnp.int32),
            pltpu.VMEM((16,), jnp.int32),
            pltpu.VMEM_SHARED((NPAD,), jnp.float32),
        ],
    )(_sc_hist_body)


def _sc_hist(dst):
    return _sc_hist_kernel()(dst)


def _sc_hist_body(dst_hbm, out_hbm, zbuf, ones_v, ones16_v, idx_v, idx16_v, acc):
    c = lax.axis_index("c")
    s = lax.axis_index("s")
    wid = s * 2 + c

    def _fill(i, _):
        zbuf[pl.ds(i * 16, 16)] = jnp.zeros((16,), jnp.float32)
        return 0
    lax.fori_loop(0, 40, _fill, 0)
    for k in range(8):
        ones_v[pl.ds(k * 16, 16)] = jnp.ones((16,), jnp.float32)
    ones16_v[...] = jnp.ones((16,), jnp.float32)

    pltpu.sync_copy(zbuf, acc.at[pl.ds(s * 640, 640)])
    plsc.subcore_barrier()

    base = wid * EPS_HIST
    def _chunk(j, _):
        pltpu.sync_copy(dst_hbm.at[pl.ds(base + j * 128, 128)], idx_v)
        pltpu.sync_copy(ones_v, acc.at[idx_v], add=True)
        return 0
    lax.fori_loop(0, HIST_CHUNKS, _chunk, 0)
    pltpu.sync_copy(dst_hbm.at[pl.ds(base + HIST_CHUNKS * 128, 16)], idx16_v)
    pltpu.sync_copy(ones16_v, acc.at[idx16_v], add=True)
    plsc.subcore_barrier()

    pltpu.sync_copy(acc.at[pl.ds(s * 640, 640)], zbuf)
    pltpu.sync_copy(zbuf, out_hbm.at[pl.ds(c * NPAD + s * 640, 640)])


GEDGES = 256
NGROUPS = E // GEDGES
GPT = 78
PR = 5120


@functools.lru_cache(maxsize=None)
def _sc_agg_kernel():
    mesh = plsc.VectorSubcoreMesh(core_axis_name="c", subcore_axis_name="s")
    return functools.partial(
        pl.kernel, mesh=mesh,
        out_type=jax.ShapeDtypeStruct((2 * N, HALF), jnp.float32),
        scratch_types=[
            pltpu.VMEM((64, HALF), jnp.float32),
            pltpu.VMEM((2, 128, HALF), jnp.float32),
            pltpu.VMEM((3, 2, 128), jnp.int32),
            pltpu.VMEM_SHARED((NPAD, HALF), jnp.float32),
            pltpu.SemaphoreType.DMA,
            pltpu.SemaphoreType.DMA,
        ],
    )(_sc_agg_body)


def _sc_agg(hcat, edges3):
    return _sc_agg_kernel()(hcat, edges3)


def _sc_agg_body(hcat_hbm, edges_hbm, out_hbm,
                 stage, msg2, ed2, acc, sem_g, sem_s):
    c = lax.axis_index("c")
    s = lax.axis_index("s")
    rbase = s * 640
    gbase = s * GPT
    cn = c * N

    def _mv(off, nrows, into_acc):
        if into_acc:
            pltpu.sync_copy(hcat_hbm.at[pl.ds(cn + off, nrows)],
                            stage.at[pl.ds(0, nrows)])
            pltpu.sync_copy(stage.at[pl.ds(0, nrows)],
                            acc.at[pl.ds(off, nrows)])
        else:
            pltpu.sync_copy(acc.at[pl.ds(off, nrows)],
                            stage.at[pl.ds(0, nrows)])
            pltpu.sync_copy(stage.at[pl.ds(0, nrows)],
                            out_hbm.at[pl.ds(cn + off, nrows)])

    def _copy_stripe(into_acc):
        @pl.when(s < 15)
        def _():
            def _full(k, _):
                _mv(rbase + k * 64, 64, into_acc)
                return 0
            lax.fori_loop(0, 10, _full, 0)

        @pl.when(s == 15)
        def _():
            def _full(k, _):
                _mv(rbase + k * 64, 64, into_acc)
                return 0
            lax.fori_loop(0, 6, _full, 0)
            _mv(rbase + 384, 16, into_acc)

    _copy_stripe(True)
    plsc.subcore_barrier()

    nchunks = jnp.where(s < 4, 157, 156)

    def _chunk_row(j):
        return c * 2500 + jnp.where(j < 156, s * 156 + j, 2496 + s)

    pltpu.sync_copy(edges_hbm.at[_chunk_row(0)], ed2.at[0])

    def _chunk(j, _):
        p = lax.rem(j, 2)
        e3 = lax.rem(j, 3)
        g = pltpu.async_copy(hcat_hbm.at[ed2.at[e3, 0]], msg2.at[p], sem_g)

        @pl.when(j + 1 < nchunks)
        def _():
            pltpu.sync_copy(edges_hbm.at[_chunk_row(j + 1)],
                            ed2.at[lax.rem(j + 1, 3)])

        @pl.when(j >= 1)
        def _():
            pltpu.make_async_copy(hcat_hbm.at[pl.ds(0, 128)],
                                  msg2.at[0], sem_s).wait()

        g.wait()
        pltpu.async_copy(msg2.at[p], acc.at[ed2.at[e3, 1]], sem_s, add=True)
        return 0
    lax.fori_loop(0, nchunks, _chunk, 0)

    pltpu.make_async_copy(hcat_hbm.at[pl.ds(0, 128)], msg2.at[0], sem_s).wait()

    plsc.subcore_barrier()
    _copy_stripe(False)


def _edges_body(s_ref, d_ref, out_ref):
    cc = pl.program_id(0)
    out_ref[...] = jnp.stack([s_ref[...] + cc * N, d_ref[...]], axis=1)


def _edges_prep(src2d, dst2d):
    nr = E // 128
    return pl.pallas_call(
        _edges_body,
        grid=(2,),
        in_specs=[
            pl.BlockSpec((nr, 128), lambda i: (0, 0)),
            pl.BlockSpec((nr, 128), lambda i: (0, 0)),
        ],
        out_specs=pl.BlockSpec((nr, 2, 128), lambda i: (i, 0, 0)),
        out_shape=jax.ShapeDtypeStruct((2 * nr, 2, 128), jnp.int32),
    )(src2d, dst2d)


def _mm_body(x_ref, w_ref, deg_ref, out_ref):
    dinv = lax.rsqrt(deg_ref[...])
    out_ref[...] = jnp.dot(x_ref[...], w_ref[...],
                           preferred_element_type=jnp.float32) * dinv


def _mm(xin, w, degcol):
    k = xin.shape[1]
    return pl.pallas_call(
        _mm_body,
        grid=(20,),
        in_specs=[
            pl.BlockSpec((ROWB, k), lambda i: (i % 10, 0)),
            pl.BlockSpec((k, HALF), lambda i: (0, i // 10)),
            pl.BlockSpec((ROWB, 1), lambda i: (i % 10, 0)),
        ],
        out_specs=pl.BlockSpec((ROWB, HALF), lambda i: (i, 0)),
        out_shape=jax.ShapeDtypeStruct((2 * N, HALF), jnp.float32),
    )(xin, w, degcol)


def _stats_body(a0_ref, a1_ref, deg_ref, b_ref, gpre_ref, s1_ref, s2_ref):
    dinv = lax.rsqrt(deg_ref[...])
    g = jnp.concatenate([a0_ref[...], a1_ref[...]], axis=1) * dinv + b_ref[...]
    gpre_ref[...] = g

    @pl.when(pl.program_id(0) == 0)
    def _():
        s1_ref[...] = jnp.zeros_like(s1_ref)
        s2_ref[...] = jnp.zeros_like(s2_ref)

    s1_ref[...] += jnp.sum(g, axis=0, keepdims=True)
    s2_ref[...] += jnp.sum(g * g, axis=0, keepdims=True)


def _stats(accf, degcol, brow):
    return pl.pallas_call(
        _stats_body,
        grid=(10,),
        in_specs=[
            pl.BlockSpec((ROWB, HALF), lambda i: (i, 0)),
            pl.BlockSpec((ROWB, HALF), lambda i: (i + 10, 0)),
            pl.BlockSpec((ROWB, 1), lambda i: (i, 0)),
            pl.BlockSpec((1, H), lambda i: (0, 0)),
        ],
        out_specs=[
            pl.BlockSpec((ROWB, H), lambda i: (i, 0)),
            pl.BlockSpec((1, H), lambda i: (0, 0)),
            pl.BlockSpec((1, H), lambda i: (0, 0)),
        ],
        out_shape=[
            jax.ShapeDtypeStruct((N, H), jnp.float32),
            jax.ShapeDtypeStruct((1, H), jnp.float32),
            jax.ShapeDtypeStruct((1, H), jnp.float32),
        ],
    )(accf, accf, degcol, brow)


def _norm_body(g_ref, s1_ref, s2_ref, ga_ref, be_ref, prev_ref, rs_ref,
               out_ref):
    m = s1_ref[...] * (1.0 / N)
    v = s2_ref[...] * (1.0 / N) - m * m
    rstd = lax.rsqrt(v + 1e-5)
    y = (g_ref[...] - m) * rstd * ga_ref[...] + be_ref[...]
    y = jnp.maximum(y, 0.0)
    out_ref[...] = y + prev_ref[...] * rs_ref[...]


def _norm(gpre, s1, s2, garow, berow, xprev, rscale):
    return pl.pallas_call(
        _norm_body,
        grid=(10,),
        in_specs=[
            pl.BlockSpec((ROWB, H), lambda i: (i, 0)),
            pl.BlockSpec((1, H), lambda i: (0, 0)),
            pl.BlockSpec((1, H), lambda i: (0, 0)),
            pl.BlockSpec((1, H), lambda i: (0, 0)),
            pl.BlockSpec((1, H), lambda i: (0, 0)),
            pl.BlockSpec((ROWB, H), lambda i: (i, 0)),
            pl.BlockSpec((1, 1), lambda i: (0, 0)),
        ],
        out_specs=pl.BlockSpec((ROWB, H), lambda i: (i, 0)),
        out_shape=jax.ShapeDtypeStruct((N, H), jnp.float32),
    )(gpre, s1, s2, garow, berow, xprev, rscale)


def _gelu(x):
    return 0.5 * x * (1.0 + lax.erf(x * 0.7071067811865476))


def _pool1_body(x3_ref, b_ref, gw1_ref, gb1_ref, gw2_ref, gb2_ref,
                lw_ref, lb_ref,
                gate_ref, cnt_ref, s1_ref, sl_ref, gm_ref, m_ref):
    x3 = x3_ref[...]
    t = _gelu(jnp.dot(x3, gw1_ref[...],
                      preferred_element_type=jnp.float32) + gb1_ref[...])
    gate = jnp.dot(t, gw2_ref[...],
                   preferred_element_type=jnp.float32) + gb2_ref[...]
    gate_ref[...] = gate
    loc = _gelu(jnp.dot(x3, lw_ref[...],
                        preferred_element_type=jnp.float32) + lb_ref[...])

    bcol = b_ref[...]
    io = lax.broadcasted_iota(jnp.int32, (ROWB, B), 1)
    ob = bcol == io
    ohf = ob.astype(jnp.float32)
    ones_col = jnp.ones((ROWB, 1), jnp.float32)
    dn = (((0,), (0,)), ((), ()))
    cntc = lax.dot_general(ohf, ones_col, dn,
                           preferred_element_type=jnp.float32)
    s1c = lax.dot_general(ohf, x3, dn,
                          preferred_element_type=jnp.float32)
    slc = lax.dot_general(ohf, loc, dn,
                          preferred_element_type=jnp.float32)
    gmc = jnp.max(jnp.where(ob, gate, NEG_INF), axis=0, keepdims=True)

    @pl.when(pl.program_id(0) == 0)
    def _():
        cnt_ref[...] = jnp.zeros_like(cnt_ref)
        s1_ref[...] = jnp.zeros_like(s1_ref)
        sl_ref[...] = jnp.zeros_like(sl_ref)
        gm_ref[...] = jnp.full_like(gm_ref, NEG_INF)
        m_ref[...] = jnp.full_like(m_ref, NEG_INF)

    cnt_ref[...] += cntc
    s1_ref[...] += s1c
    sl_ref[...] += slc
    gm_ref[...] = jnp.maximum(gm_ref[...], gmc)

    bmin = jnp.min(bcol)
    bmax = jnp.max(bcol)
    rio = lax.broadcasted_iota(jnp.int32, (B, 1), 0)

    def _mb(bi, _):
        mask = bcol == bi
        mrow = jnp.max(jnp.where(mask, x3, NEG_INF), axis=0, keepdims=True)
        cur = m_ref[...]
        m_ref[...] = jnp.where(rio == bi, jnp.maximum(cur, mrow), cur)
        return 0

    lax.fori_loop(bmin, bmax + 1, _mb, 0)


def _pool1(x3, batchcol, gw1, gb1, gw2, gb2, lw, lb):
    return pl.pallas_call(
        _pool1_body,
        grid=(10,),
        in_specs=[
            pl.BlockSpec((ROWB, H), lambda i: (i, 0)),
            pl.BlockSpec((ROWB, 1), lambda i: (i, 0)),
            pl.BlockSpec((H, 128), lambda i: (0, 0)),
            pl.BlockSpec((1, 128), lambda i: (0, 0)),
            pl.BlockSpec((128, 1), lambda i: (0, 0)),
            pl.BlockSpec((1, 1), lambda i: (0, 0)),
            pl.BlockSpec((H, 128), lambda i: (0, 0)),
            pl.BlockSpec((1, 128), lambda i: (0, 0)),
        ],
        out_specs=[
            pl.BlockSpec((ROWB, 1), lambda i: (i, 0)),
            pl.BlockSpec((B, 1), lambda i: (0, 0)),
            pl.BlockSpec((B, H), lambda i: (0, 0)),
            pl.BlockSpec((B, 128), lambda i: (0, 0)),
            pl.BlockSpec((1, B), lambda i: (0, 0)),
            pl.BlockSpec((B, H), lambda i: (0, 0)),
        ],
        out_shape=[
            jax.ShapeDtypeStruct((N, 1), jnp.float32),
            jax.ShapeDtypeStruct((B, 1), jnp.float32),
            jax.ShapeDtypeStruct((B, H), jnp.float32),
            jax.ShapeDtypeStruct((B, 128), jnp.float32),
            jax.ShapeDtypeStruct((1, B), jnp.float32),
            jax.ShapeDtypeStruct((B, H), jnp.float32),
        ],
    )(x3, batchcol, gw1, gb1, gw2, gb2, lw, lb)


def _pool2_body(x3_ref, gate_ref, b_ref, gm_ref, den_ref, z_ref):
    x3 = x3_ref[...]
    gate = gate_ref[...]
    bcol = b_ref[...]
    io = lax.broadcasted_iota(jnp.int32, (ROWB, B), 1)
    ob = bcol == io
    ohf = ob.astype(jnp.float32)
    gmb = jnp.sum(jnp.where(ob, gm_ref[...], 0.0), axis=1, keepdims=True)
    e = jnp.exp(gate - gmb)
    dn = (((0,), (0,)), ((), ()))
    denc = lax.dot_general(ohf, e, dn,
                           preferred_element_type=jnp.float32)
    zc = lax.dot_general(ohf * e, x3, dn,
                         preferred_element_type=jnp.float32)

    @pl.when(pl.program_id(0) == 0)
    def _():
        den_ref[...] = jnp.zeros_like(den_ref)
        z_ref[...] = jnp.zeros_like(z_ref)

    den_ref[...] += denc
    z_ref[...] += zc


def _pool2(x3, gate, batchcol, gm):
    return pl.pallas_call(
        _pool2_body,
        grid=(10,),
        in_specs=[
            pl.BlockSpec((ROWB, H), lambda i: (i, 0)),
            pl.BlockSpec((ROWB, 1), lambda i: (i, 0)),
            pl.BlockSpec((ROWB, 1), lambda i: (i, 0)),
            pl.BlockSpec((1, B), lambda i: (0, 0)),
        ],
        out_specs=[
            pl.BlockSpec((B, 1), lambda i: (0, 0)),
            pl.BlockSpec((B, H), lambda i: (0, 0)),
        ],
        out_shape=[
            jax.ShapeDtypeStruct((B, 1), jnp.float32),
            jax.ShapeDtypeStruct((B, H), jnp.float32),
        ],
    )(x3, gate, batchcol, gm)


def _head_body(cnt_ref, s1_ref, m_ref, z_ref, den_ref, sl_ref, adme_ref,
               w1_ref, b1_ref, w2_ref, b2_ref, w3_ref, b3_ref,
               w4_ref, b4_ref, w5_ref, b5_ref, out_ref, comb):
    c = jnp.maximum(cnt_ref[...], 1.0)
    comb[:, 0:256] = s1_ref[...] / c
    comb[:, 256:512] = m_ref[...]
    comb[:, 512:768] = z_ref[...] / den_ref[...]
    comb[:, 768:896] = sl_ref[...] / c
    comb[:, 896:1024] = jnp.concatenate(
        [adme_ref[...], jnp.zeros((B, 98), jnp.float32)], axis=1)
    h = comb[...]
    h = jnp.maximum(jnp.dot(h, w1_ref[...],
                            preferred_element_type=jnp.float32)
                    + b1_ref[...], 0.0)
    h = jnp.maximum(jnp.dot(h, w2_ref[...],
                            preferred_element_type=jnp.float32)
                    + b2_ref[...], 0.0)
    h = jnp.maximum(jnp.dot(h, w3_ref[...],
                            preferred_element_type=jnp.float32)
                    + b3_ref[...], 0.0)
    h = jnp.maximum(jnp.dot(h, w4_ref[...],
                            preferred_element_type=jnp.float32)
                    + b4_ref[...], 0.0)
    out_ref[...] = jnp.dot(h, w5_ref[...],
                           preferred_element_type=jnp.float32) + b5_ref[...]


def _head(cnt, s1, m, z, den, sl, adme, w1p, b1, w2, b2, w3, b3, w4, b4,
          w5, b5):
    return pl.pallas_call(
        _head_body,
        out_shape=jax.ShapeDtypeStruct((B, 1), jnp.float32),
        scratch_shapes=[pltpu.VMEM((B, 1024), jnp.float32)],
    )(cnt, s1, m, z, den, sl, adme, w1p, b1, w2, b2, w3, b3, w4, b4, w5, b5)


def kernel(x, edge_index, batch, adme_features, W1, b1, W2, b2, W3, b3,
           g1, be1, g2, be2, g3, be3, gW1, gb1, gW2, gb2, lW, lb,
           hW1, hb1, hW2, hb2, hW3, hb3, hW4, hb4, hW5, hb5):
    src = edge_index[0]
    dst = edge_index[1]
    edges3 = _edges_prep(src.reshape(E // 128, 128),
                         dst.reshape(E // 128, 128))

    hist = _sc_hist(dst)
    degcol = (hist[:NPAD][:N] + hist[NPAD:][:N] + 1.0).reshape(N, 1)

    batchcol = batch.reshape(N, 1)
    row = lambda v: v.reshape(1, -1)

    wst = jnp.stack([jnp.pad(W1, ((0, H - D_IN), (0, 0))), W2, W3])
    bst = jnp.stack([b1, b2, b3]).reshape(3, 1, H)
    gst = jnp.stack([g1, g2, g3]).reshape(3, 1, H)
    best = jnp.stack([be1, be2, be3]).reshape(3, 1, H)
    rst = jnp.array([0.0, 1.0, 1.0], jnp.float32).reshape(3, 1, 1)
    x0 = jnp.pad(x, ((0, 0), (0, H - D_IN)))

    def _layer(l, xc):
        w = wst[l]
        brow = bst[l]
        garow = gst[l]
        berow = best[l]
        rs = rst[l]
        hcat = _mm(xc, w, degcol)
        accf = _sc_agg(hcat, edges3)
        gpre, s1, s2 = _stats(accf, degcol, brow)
        return _norm(gpre, s1, s2, garow, berow, xc, rs)

    xcur = x0
    for l in range(3):
        xcur = _layer(l, xcur)

    gate, cnt, s1p, slp, gm, mp = _pool1(
        xcur, batchcol, gW1, row(gb1), gW2, row(gb2), lW, row(lb))
    den, zp = _pool2(xcur, gate, batchcol, gm)

    w1p = jnp.pad(hW1, ((0, 1024 - hW1.shape[0]), (0, 0)))
    out = _head(cnt, s1p, mp, zp, den, slp, adme_features,
                w1p, row(hb1), hW2, row(hb2), hW3, row(hb3),
                hW4, row(hb4), hW5, row(hb5))
    return out[:, 0]
